# baseline (speedup 1.0000x reference)
import functools

import jax
import jax.numpy as jnp
from jax import lax
from jax.experimental import pallas as pl
from jax.experimental.pallas import tpu as pltpu
from jax.experimental.pallas import tpu_sc as plsc

N_NODES = 10000
N_EDGES = 160000
UNITS = 64
NUM_RBF = 32
CUTOFF = 5.0


EDGE_BLK = 3200
EW_ROWS = EDGE_BLK // 128


def _edge_mlp_body(attr_ref, ew_ref, w0_ref, b0_ref, w1_ref, b1_ref,
                   w2_ref, b2_ref, h_ref):
    a = attr_ref[...].astype(jnp.bfloat16)
    h1 = jax.nn.silu(
        jnp.dot(a, w0_ref[...], preferred_element_type=jnp.float32)
        + b0_ref[...])
    h2 = jax.nn.silu(
        jnp.dot(h1.astype(jnp.bfloat16), w1_ref[...],
                preferred_element_type=jnp.float32)
        + b1_ref[...])
    h3 = jax.nn.silu(
        jnp.dot(h2.astype(jnp.bfloat16), w2_ref[...],
                preferred_element_type=jnp.float32)
        + b2_ref[...])
    r = ew_ref[0]
    c = 0.5 * (jnp.cos(r * (jnp.pi / CUTOFF)) + 1.0)
    c = jnp.where(r < CUTOFF, c, 0.0)
    prod = h3.reshape(EW_ROWS, 128, 3 * UNITS) * c[:, :, None]
    prod = prod.reshape(EDGE_BLK, 3 * UNITS)
    h_ref[...] = jnp.concatenate(
        [prod, jnp.zeros((EDGE_BLK, UNITS), jnp.float32)], axis=1)


def _edge_mlp(edge_attr, edge_weight, w0, b0, w1, b1, w2p, b2p):
    grid = N_EDGES // EDGE_BLK
    return pl.pallas_call(
        _edge_mlp_body,
        grid=(grid,),
        in_specs=[
            pl.BlockSpec((EDGE_BLK, NUM_RBF), lambda i: (i, 0)),
            pl.BlockSpec((1, EW_ROWS, 128), lambda i: (i, 0, 0)),
            pl.BlockSpec((NUM_RBF, UNITS), lambda i: (0, 0)),
            pl.BlockSpec((1, UNITS), lambda i: (0, 0)),
            pl.BlockSpec((UNITS, 2 * UNITS), lambda i: (0, 0)),
            pl.BlockSpec((1, 2 * UNITS), lambda i: (0, 0)),
            pl.BlockSpec((2 * UNITS, 3 * UNITS), lambda i: (0, 0)),
            pl.BlockSpec((1, 3 * UNITS), lambda i: (0, 0)),
        ],
        out_specs=pl.BlockSpec((EDGE_BLK, 4 * UNITS), lambda i: (i, 0)),
        out_shape=jax.ShapeDtypeStruct((N_EDGES, 4 * UNITS), jnp.float32),
    )(edge_attr, edge_weight, w0, b0, w1, b1, w2p, b2p)


COLS_PER_CORE = 128
EPW = N_EDGES // 16
CHUNK = 128
NFULL = EPW // CHUNK
TAIL = EPW - NFULL * CHUNK
N_ACC = 10240
ROWS_PER_SUB = N_ACC // 16
LAST_ROWS = N_NODES - 15 * ROWS_PER_SUB


def _sc_segsum_body(h_hbm, dst_hbm, zeros_hbm, out_hbm,
                    idx0_v, idx1_v, rows0_v, rows1_v, idxt_v, rowst_v, acc_sh,
                    sem_i0, sem_i1, sem_r0, sem_r1):
    c = lax.axis_index("c")
    s = lax.axis_index("s")
    col0 = c * COLS_PER_CORE
    pltpu.sync_copy(zeros_hbm, acc_sh.at[pl.ds(s * ROWS_PER_SUB, ROWS_PER_SUB)])
    plsc.subcore_barrier()

    base = s * EPW
    idxb = (idx0_v, idx1_v)
    rowsb = (rows0_v, rows1_v)
    semi = (sem_i0, sem_i1)
    semr = (sem_r0, sem_r1)

    def start_load(chunk, b):
        off = base + chunk * CHUNK
        pltpu.async_copy(dst_hbm.at[pl.ds(off, CHUNK)], idxb[b], semi[b])
        pltpu.async_copy(
            h_hbm.at[pl.ds(off, CHUNK), pl.ds(col0, COLS_PER_CORE)],
            rowsb[b], semr[b])

    def wait_load(b):
        pltpu.make_async_copy(dst_hbm.at[pl.ds(0, CHUNK)], idxb[b],
                              semi[b]).wait()
        pltpu.make_async_copy(
            h_hbm.at[pl.ds(0, CHUNK), pl.ds(col0, COLS_PER_CORE)],
            rowsb[b], semr[b]).wait()

    start_load(0, 0)
    start_load(1, 1)

    def chunk_body(i, carry):
        for k in (0, 1):
            g = 2 * i + k
            wait_load(k)
            pltpu.sync_copy(rowsb[k], acc_sh.at[idxb[k]], add=True)

            @pl.when(g + 2 < NFULL)
            def _():
                start_load(g + 2, k)
        return carry

    lax.fori_loop(0, NFULL // 2, chunk_body, 0)

    b = base + NFULL * CHUNK
    pltpu.sync_copy(dst_hbm.at[pl.ds(b, TAIL)], idxt_v)
    pltpu.sync_copy(h_hbm.at[pl.ds(b, TAIL), pl.ds(col0, COLS_PER_CORE)],
                    rowst_v)
    pltpu.sync_copy(rowst_v, acc_sh.at[idxt_v], add=True)

    plsc.subcore_barrier()

    @pl.when(s < 15)
    def _():
        pltpu.sync_copy(acc_sh.at[pl.ds(s * ROWS_PER_SUB, ROWS_PER_SUB)],
                        out_hbm.at[pl.ds(s * ROWS_PER_SUB, ROWS_PER_SUB),
                                   pl.ds(col0, COLS_PER_CORE)])

    @pl.when(s == 15)
    def _():
        pltpu.sync_copy(acc_sh.at[pl.ds(15 * ROWS_PER_SUB, LAST_ROWS)],
                        out_hbm.at[pl.ds(15 * ROWS_PER_SUB, LAST_ROWS),
                                   pl.ds(col0, COLS_PER_CORE)])


def _sc_segsum(h, dst):
    zeros = jnp.zeros((ROWS_PER_SUB, COLS_PER_CORE), jnp.float32)
    mesh = plsc.VectorSubcoreMesh(core_axis_name="c", subcore_axis_name="s")
    f = pl.kernel(
        _sc_segsum_body,
        out_type=jax.ShapeDtypeStruct((N_NODES, 4 * UNITS), jnp.float32),
        mesh=mesh,
        scratch_types=[
            pltpu.VMEM((CHUNK,), jnp.int32),
            pltpu.VMEM((CHUNK,), jnp.int32),
            pltpu.VMEM((CHUNK, COLS_PER_CORE), jnp.float32),
            pltpu.VMEM((CHUNK, COLS_PER_CORE), jnp.float32),
            pltpu.VMEM((TAIL,), jnp.int32),
            pltpu.VMEM((TAIL, COLS_PER_CORE), jnp.float32),
            pltpu.VMEM_SHARED((N_ACC, COLS_PER_CORE), jnp.float32),
            pltpu.SemaphoreType.DMA,
            pltpu.SemaphoreType.DMA,
            pltpu.SemaphoreType.DMA,
            pltpu.SemaphoreType.DMA,
        ],
    )
    return f(h, dst, zeros)


NODE_BLK = 400

_IJ = [(0, 0), (0, 1), (0, 2), (1, 0), (1, 1), (1, 2), (2, 0), (2, 1), (2, 2)]
_UPPER = [(0, 1), (0, 2), (1, 2)]
_DIAG = [(0, 0), (1, 1), (2, 2)]
_SYM = [(0, 0), (0, 1), (0, 2), (1, 1), (1, 2), (2, 2)]


def _node_body(x_ref, h_ref, wt0_ref, wt1_ref, wt2_ref, wt3_ref, wt4_ref,
               wt5_ref, o_ref):
    xf = x_ref[...]
    x = [[xf[:, (3 * i + j) * UNITS:(3 * i + j + 1) * UNITS]
          for j in range(3)] for i in range(3)]
    norm = 1.0 + sum(x[i][j] * x[i][j] for (i, j) in _IJ)
    rn = 1.0 / norm
    xn = [[x[i][j] * rn for j in range(3)] for i in range(3)]

    i0 = (xn[0][0] + xn[1][1] + xn[2][2]) * (1.0 / 3.0)
    a = {(i, j): 0.5 * (xn[i][j] - xn[j][i]) for (i, j) in _UPPER}
    s = {(i, j): 0.5 * (xn[i][j] + xn[j][i]) for (i, j) in _UPPER}
    for (i, j) in _DIAG:
        s[(i, j)] = xn[i][j] - i0

    wt0 = wt0_ref[...]
    wt1 = wt1_ref[...]
    wt2 = wt2_ref[...]

    def mm(v, w):
        return jnp.dot(v, w, preferred_element_type=jnp.float32)

    il = mm(i0, wt0)
    al = {k: mm(a[k], wt1) for k in _UPPER}
    sl = {k: mm(s[k], wt2) for k in _SYM}

    def full_mat(diag_i, ad, sd):
        m = [[None] * 3 for _ in range(3)]
        for i in range(3):
            for j in range(3):
                if i == j:
                    m[i][j] = diag_i + sd[(i, i)]
                elif i < j:
                    m[i][j] = ad[(i, j)] + sd[(i, j)]
                else:
                    m[i][j] = sd[(j, i)] - ad[(j, i)]
        return m

    y = full_mat(il, al, sl)

    hsum = h_ref[...]
    h0 = hsum[:, 0:UNITS]
    h1 = hsum[:, UNITS:2 * UNITS]
    h2 = hsum[:, 2 * UNITS:3 * UNITS]

    ilh = il * h0
    alh = {k: al[k] * h1 for k in _UPPER}
    slh = {k: sl[k] * h2 for k in _SYM}
    msg = full_mat(ilh, alh, slh)

    c2 = [[None] * 3 for _ in range(3)]
    for i in range(3):
        for l in range(3):
            acc = y[i][0] * msg[0][l] + msg[i][0] * y[0][l]
            acc += y[i][1] * msg[1][l] + msg[i][1] * y[1][l]
            acc += y[i][2] * msg[2][l] + msg[i][2] * y[2][l]
            c2[i][l] = acc

    i2 = (c2[0][0] + c2[1][1] + c2[2][2]) * (1.0 / 3.0)
    np1 = 1.0 + sum(c2[i][j] * c2[i][j] for (i, j) in _IJ)
    rp = 1.0 / np1

    a2 = {(i, j): (0.5 * (c2[i][j] - c2[j][i])) * rp for (i, j) in _UPPER}
    s2 = {(i, j): (0.5 * (c2[i][j] + c2[j][i])) * rp for (i, j) in _UPPER}
    for (i, j) in _DIAG:
        s2[(i, j)] = (c2[i][j] - i2) * rp
    i2r = i2 * rp

    i3 = mm(i2r, wt3_ref[...])
    wt4 = wt4_ref[...]
    wt5 = wt5_ref[...]
    a3 = {k: mm(a2[k], wt4) for k in _UPPER}
    s3 = {k: mm(s2[k], wt5) for k in _SYM}
    dx = full_mat(i3, a3, s3)

    ocols = []
    for i in range(3):
        for j in range(3):
            o = xn[i][j] + dx[i][j]
            o += dx[i][0] * dx[0][j] + dx[i][1] * dx[1][j] + dx[i][2] * dx[2][j]
            ocols.append(o)
    o_ref[...] = jnp.concatenate(ocols, axis=1)


def _node_stage(x, hsum, wt0, wt1, wt2, wt3, wt4, wt5):
    grid = N_NODES // NODE_BLK
    wspec = pl.BlockSpec((UNITS, UNITS), lambda i: (0, 0))
    return pl.pallas_call(
        _node_body,
        grid=(grid,),
        in_specs=[
            pl.BlockSpec((NODE_BLK, 9 * UNITS), lambda i: (i, 0)),
            pl.BlockSpec((NODE_BLK, 4 * UNITS), lambda i: (i, 0)),
            wspec, wspec, wspec, wspec, wspec, wspec,
        ],
        out_specs=pl.BlockSpec((NODE_BLK, 9 * UNITS), lambda i: (i, 0)),
        out_shape=jax.ShapeDtypeStruct((N_NODES, 9 * UNITS), jnp.float32),
    )(x, hsum, wt0, wt1, wt2, wt3, wt4, wt5)


def kernel(X, edge_index, edge_weight, edge_attr, Ws0, bs0, Ws1, bs1, Ws2,
           bs2, Wt0, Wt1, Wt2, Wt3, Wt4, Wt5):
    perm = (jnp.arange(3)[:, None] + 3 * jnp.arange(UNITS)[None, :]).reshape(-1)
    w2p = Ws2[:, perm]
    b2p = bs2[perm]

    h = _edge_mlp(edge_attr, edge_weight.reshape(-1, EW_ROWS, 128),
                  Ws0.astype(jnp.bfloat16), bs0.reshape(1, -1),
                  Ws1.astype(jnp.bfloat16), bs1.reshape(1, -1),
                  w2p.astype(jnp.bfloat16), b2p.reshape(1, -1))
    dst = edge_index[1]
    xb, h = lax.optimization_barrier((X, h))
    hsum = _sc_segsum(h, dst)
    out = _node_stage(xb.reshape(N_NODES, 9 * UNITS), hsum,
                      Wt0, Wt1, Wt2, Wt3, Wt4, Wt5)
    return out.reshape(N_NODES, 3, 3, UNITS)


# speedup vs baseline: 1.0127x; 1.0127x over previous
import functools

import jax
import jax.numpy as jnp
from jax import lax
from jax.experimental import pallas as pl
from jax.experimental.pallas import tpu as pltpu
from jax.experimental.pallas import tpu_sc as plsc

N_NODES = 10000
N_EDGES = 160000
UNITS = 64
NUM_RBF = 32
CUTOFF = 5.0


EDGE_BLK = 6400
EW_ROWS = EDGE_BLK // 128


def _edge_mlp_body(attr_ref, ew_ref, w0_ref, b0_ref, w1_ref, b1_ref,
                   w2_ref, b2_ref, h_ref):
    a = attr_ref[...].astype(jnp.bfloat16)
    h1 = jax.nn.silu(
        jnp.dot(a, w0_ref[...], preferred_element_type=jnp.float32)
        + b0_ref[...])
    h2 = jax.nn.silu(
        jnp.dot(h1.astype(jnp.bfloat16), w1_ref[...],
                preferred_element_type=jnp.float32)
        + b1_ref[...])
    h3 = jax.nn.silu(
        jnp.dot(h2.astype(jnp.bfloat16), w2_ref[...],
                preferred_element_type=jnp.float32)
        + b2_ref[...])
    r = ew_ref[0]
    c = 0.5 * (jnp.cos(r * (jnp.pi / CUTOFF)) + 1.0)
    c = jnp.where(r < CUTOFF, c, 0.0)
    prod = h3.reshape(EW_ROWS, 128, 3 * UNITS) * c[:, :, None]
    prod = prod.reshape(EDGE_BLK, 3 * UNITS)
    h_ref[...] = jnp.concatenate(
        [prod, jnp.zeros((EDGE_BLK, UNITS), jnp.float32)], axis=1)


def _edge_mlp(edge_attr, edge_weight, w0, b0, w1, b1, w2p, b2p):
    grid = N_EDGES // EDGE_BLK
    return pl.pallas_call(
        _edge_mlp_body,
        grid=(grid,),
        in_specs=[
            pl.BlockSpec((EDGE_BLK, NUM_RBF), lambda i: (i, 0)),
            pl.BlockSpec((1, EW_ROWS, 128), lambda i: (i, 0, 0)),
            pl.BlockSpec((NUM_RBF, UNITS), lambda i: (0, 0)),
            pl.BlockSpec((1, UNITS), lambda i: (0, 0)),
            pl.BlockSpec((UNITS, 2 * UNITS), lambda i: (0, 0)),
            pl.BlockSpec((1, 2 * UNITS), lambda i: (0, 0)),
            pl.BlockSpec((2 * UNITS, 3 * UNITS), lambda i: (0, 0)),
            pl.BlockSpec((1, 3 * UNITS), lambda i: (0, 0)),
        ],
        out_specs=pl.BlockSpec((EDGE_BLK, 4 * UNITS), lambda i: (i, 0)),
        out_shape=jax.ShapeDtypeStruct((N_EDGES, 4 * UNITS), jnp.float32),
    )(edge_attr, edge_weight, w0, b0, w1, b1, w2p, b2p)


COLS_PER_CORE = 128
EPW = N_EDGES // 16
CHUNK = 128
NFULL = EPW // CHUNK
TAIL = EPW - NFULL * CHUNK
N_ACC = 10240
ROWS_PER_SUB = N_ACC // 16
LAST_ROWS = N_NODES - 15 * ROWS_PER_SUB


def _sc_segsum_body(h_hbm, dst_hbm, zeros_hbm, out_hbm,
                    idx0_v, idx1_v, rows0_v, rows1_v, idxt_v, rowst_v, acc_sh,
                    sem_i0, sem_i1, sem_r0, sem_r1):
    c = lax.axis_index("c")
    s = lax.axis_index("s")
    col0 = c * COLS_PER_CORE
    pltpu.sync_copy(zeros_hbm, acc_sh.at[pl.ds(s * ROWS_PER_SUB, ROWS_PER_SUB)])
    plsc.subcore_barrier()

    base = s * EPW
    idxb = (idx0_v, idx1_v)
    rowsb = (rows0_v, rows1_v)
    semi = (sem_i0, sem_i1)
    semr = (sem_r0, sem_r1)

    def start_load(chunk, b):
        off = base + chunk * CHUNK
        pltpu.async_copy(dst_hbm.at[pl.ds(off, CHUNK)], idxb[b], semi[b])
        pltpu.async_copy(
            h_hbm.at[pl.ds(off, CHUNK), pl.ds(col0, COLS_PER_CORE)],
            rowsb[b], semr[b])

    def wait_load(b):
        pltpu.make_async_copy(dst_hbm.at[pl.ds(0, CHUNK)], idxb[b],
                              semi[b]).wait()
        pltpu.make_async_copy(
            h_hbm.at[pl.ds(0, CHUNK), pl.ds(col0, COLS_PER_CORE)],
            rowsb[b], semr[b]).wait()

    start_load(0, 0)
    start_load(1, 1)

    def chunk_body(i, carry):
        for k in (0, 1):
            g = 2 * i + k
            wait_load(k)
            pltpu.sync_copy(rowsb[k], acc_sh.at[idxb[k]], add=True)

            @pl.when(g + 2 < NFULL)
            def _():
                start_load(g + 2, k)
        return carry

    lax.fori_loop(0, NFULL // 2, chunk_body, 0)

    b = base + NFULL * CHUNK
    pltpu.sync_copy(dst_hbm.at[pl.ds(b, TAIL)], idxt_v)
    pltpu.sync_copy(h_hbm.at[pl.ds(b, TAIL), pl.ds(col0, COLS_PER_CORE)],
                    rowst_v)
    pltpu.sync_copy(rowst_v, acc_sh.at[idxt_v], add=True)

    plsc.subcore_barrier()

    @pl.when(s < 15)
    def _():
        pltpu.sync_copy(acc_sh.at[pl.ds(s * ROWS_PER_SUB, ROWS_PER_SUB)],
                        out_hbm.at[pl.ds(s * ROWS_PER_SUB, ROWS_PER_SUB),
                                   pl.ds(col0, COLS_PER_CORE)])

    @pl.when(s == 15)
    def _():
        pltpu.sync_copy(acc_sh.at[pl.ds(15 * ROWS_PER_SUB, LAST_ROWS)],
                        out_hbm.at[pl.ds(15 * ROWS_PER_SUB, LAST_ROWS),
                                   pl.ds(col0, COLS_PER_CORE)])


def _sc_segsum(h, dst):
    zeros = jnp.zeros((ROWS_PER_SUB, COLS_PER_CORE), jnp.float32)
    mesh = plsc.VectorSubcoreMesh(core_axis_name="c", subcore_axis_name="s")
    f = pl.kernel(
        _sc_segsum_body,
        out_type=jax.ShapeDtypeStruct((N_NODES, 4 * UNITS), jnp.float32),
        mesh=mesh,
        scratch_types=[
            pltpu.VMEM((CHUNK,), jnp.int32),
            pltpu.VMEM((CHUNK,), jnp.int32),
            pltpu.VMEM((CHUNK, COLS_PER_CORE), jnp.float32),
            pltpu.VMEM((CHUNK, COLS_PER_CORE), jnp.float32),
            pltpu.VMEM((TAIL,), jnp.int32),
            pltpu.VMEM((TAIL, COLS_PER_CORE), jnp.float32),
            pltpu.VMEM_SHARED((N_ACC, COLS_PER_CORE), jnp.float32),
            pltpu.SemaphoreType.DMA,
            pltpu.SemaphoreType.DMA,
            pltpu.SemaphoreType.DMA,
            pltpu.SemaphoreType.DMA,
        ],
    )
    return f(h, dst, zeros)


NODE_BLK = 400

_IJ = [(0, 0), (0, 1), (0, 2), (1, 0), (1, 1), (1, 2), (2, 0), (2, 1), (2, 2)]
_UPPER = [(0, 1), (0, 2), (1, 2)]
_DIAG = [(0, 0), (1, 1), (2, 2)]
_SYM = [(0, 0), (0, 1), (0, 2), (1, 1), (1, 2), (2, 2)]


def _node_body(x_ref, h_ref, wt0_ref, wt1_ref, wt2_ref, wt3_ref, wt4_ref,
               wt5_ref, o_ref):
    xf = x_ref[...]
    x = [[xf[:, (3 * i + j) * UNITS:(3 * i + j + 1) * UNITS]
          for j in range(3)] for i in range(3)]
    norm = 1.0 + sum(x[i][j] * x[i][j] for (i, j) in _IJ)
    rn = 1.0 / norm
    xn = [[x[i][j] * rn for j in range(3)] for i in range(3)]

    i0 = (xn[0][0] + xn[1][1] + xn[2][2]) * (1.0 / 3.0)
    a = {(i, j): 0.5 * (xn[i][j] - xn[j][i]) for (i, j) in _UPPER}
    s = {(i, j): 0.5 * (xn[i][j] + xn[j][i]) for (i, j) in _UPPER}
    for (i, j) in _DIAG:
        s[(i, j)] = xn[i][j] - i0

    wt0 = wt0_ref[...]
    wt1 = wt1_ref[...]
    wt2 = wt2_ref[...]

    def mm(v, w):
        return jnp.dot(v, w, preferred_element_type=jnp.float32)

    il = mm(i0, wt0)
    al = {k: mm(a[k], wt1) for k in _UPPER}
    sl = {k: mm(s[k], wt2) for k in _SYM}

    def full_mat(diag_i, ad, sd):
        m = [[None] * 3 for _ in range(3)]
        for i in range(3):
            for j in range(3):
                if i == j:
                    m[i][j] = diag_i + sd[(i, i)]
                elif i < j:
                    m[i][j] = ad[(i, j)] + sd[(i, j)]
                else:
                    m[i][j] = sd[(j, i)] - ad[(j, i)]
        return m

    y = full_mat(il, al, sl)

    hsum = h_ref[...]
    h0 = hsum[:, 0:UNITS]
    h1 = hsum[:, UNITS:2 * UNITS]
    h2 = hsum[:, 2 * UNITS:3 * UNITS]

    ilh = il * h0
    alh = {k: al[k] * h1 for k in _UPPER}
    slh = {k: sl[k] * h2 for k in _SYM}
    msg = full_mat(ilh, alh, slh)

    c2 = [[None] * 3 for _ in range(3)]
    for i in range(3):
        for l in range(3):
            acc = y[i][0] * msg[0][l] + msg[i][0] * y[0][l]
            acc += y[i][1] * msg[1][l] + msg[i][1] * y[1][l]
            acc += y[i][2] * msg[2][l] + msg[i][2] * y[2][l]
            c2[i][l] = acc

    i2 = (c2[0][0] + c2[1][1] + c2[2][2]) * (1.0 / 3.0)
    np1 = 1.0 + sum(c2[i][j] * c2[i][j] for (i, j) in _IJ)
    rp = 1.0 / np1

    a2 = {(i, j): (0.5 * (c2[i][j] - c2[j][i])) * rp for (i, j) in _UPPER}
    s2 = {(i, j): (0.5 * (c2[i][j] + c2[j][i])) * rp for (i, j) in _UPPER}
    for (i, j) in _DIAG:
        s2[(i, j)] = (c2[i][j] - i2) * rp
    i2r = i2 * rp

    i3 = mm(i2r, wt3_ref[...])
    wt4 = wt4_ref[...]
    wt5 = wt5_ref[...]
    a3 = {k: mm(a2[k], wt4) for k in _UPPER}
    s3 = {k: mm(s2[k], wt5) for k in _SYM}
    dx = full_mat(i3, a3, s3)

    ocols = []
    for i in range(3):
        for j in range(3):
            o = xn[i][j] + dx[i][j]
            o += dx[i][0] * dx[0][j] + dx[i][1] * dx[1][j] + dx[i][2] * dx[2][j]
            ocols.append(o)
    o_ref[...] = jnp.concatenate(ocols, axis=1)


def _node_stage(x, hsum, wt0, wt1, wt2, wt3, wt4, wt5):
    grid = N_NODES // NODE_BLK
    wspec = pl.BlockSpec((UNITS, UNITS), lambda i: (0, 0))
    return pl.pallas_call(
        _node_body,
        grid=(grid,),
        in_specs=[
            pl.BlockSpec((NODE_BLK, 9 * UNITS), lambda i: (i, 0)),
            pl.BlockSpec((NODE_BLK, 4 * UNITS), lambda i: (i, 0)),
            wspec, wspec, wspec, wspec, wspec, wspec,
        ],
        out_specs=pl.BlockSpec((NODE_BLK, 9 * UNITS), lambda i: (i, 0)),
        out_shape=jax.ShapeDtypeStruct((N_NODES, 9 * UNITS), jnp.float32),
    )(x, hsum, wt0, wt1, wt2, wt3, wt4, wt5)


def kernel(X, edge_index, edge_weight, edge_attr, Ws0, bs0, Ws1, bs1, Ws2,
           bs2, Wt0, Wt1, Wt2, Wt3, Wt4, Wt5):
    perm = (jnp.arange(3)[:, None] + 3 * jnp.arange(UNITS)[None, :]).reshape(-1)
    w2p = Ws2[:, perm]
    b2p = bs2[perm]

    h = _edge_mlp(edge_attr, edge_weight.reshape(-1, EW_ROWS, 128),
                  Ws0.astype(jnp.bfloat16), bs0.reshape(1, -1),
                  Ws1.astype(jnp.bfloat16), bs1.reshape(1, -1),
                  w2p.astype(jnp.bfloat16), b2p.reshape(1, -1))
    dst = edge_index[1]
    xb = X + h[0, 0] * 0.0
    hsum = _sc_segsum(h, dst)
    out = _node_stage(xb.reshape(N_NODES, 9 * UNITS), hsum,
                      Wt0, Wt1, Wt2, Wt3, Wt4, Wt5)
    return out.reshape(N_NODES, 3, 3, UNITS)


# speedup vs baseline: 1.0399x; 1.0269x over previous
import functools

import jax
import jax.numpy as jnp
from jax import lax
from jax.experimental import pallas as pl
from jax.experimental.pallas import tpu as pltpu
from jax.experimental.pallas import tpu_sc as plsc

N_NODES = 10000
N_EDGES = 160000
UNITS = 64
NUM_RBF = 32
CUTOFF = 5.0


EDGE_BLK = 6400
EW_ROWS = EDGE_BLK // 128


def _edge_mlp_body(attr_ref, ew_ref, w0_ref, b0_ref, w1_ref, b1_ref,
                   w2_ref, b2_ref, h_ref):
    a = attr_ref[...].astype(jnp.bfloat16)
    h1 = jax.nn.silu(
        jnp.dot(a, w0_ref[...], preferred_element_type=jnp.float32)
        + b0_ref[...])
    h2 = jax.nn.silu(
        jnp.dot(h1.astype(jnp.bfloat16), w1_ref[...],
                preferred_element_type=jnp.float32)
        + b1_ref[...])
    h3 = jax.nn.silu(
        jnp.dot(h2.astype(jnp.bfloat16), w2_ref[...],
                preferred_element_type=jnp.float32)
        + b2_ref[...])
    r = ew_ref[0]
    c = 0.5 * (jnp.cos(r * (jnp.pi / CUTOFF)) + 1.0)
    c = jnp.where(r < CUTOFF, c, 0.0)
    prod = h3.reshape(EW_ROWS, 128, 3 * UNITS) * c[:, :, None]
    prod = prod.reshape(EDGE_BLK, 3 * UNITS)
    h_ref[...] = jnp.concatenate(
        [prod, jnp.zeros((EDGE_BLK, UNITS), jnp.float32)], axis=1)


def _edge_mlp(edge_attr, edge_weight, w0, b0, w1, b1, w2p, b2p):
    grid = N_EDGES // EDGE_BLK
    return pl.pallas_call(
        _edge_mlp_body,
        grid=(grid,),
        in_specs=[
            pl.BlockSpec((EDGE_BLK, NUM_RBF), lambda i: (i, 0)),
            pl.BlockSpec((1, EW_ROWS, 128), lambda i: (i, 0, 0)),
            pl.BlockSpec((NUM_RBF, UNITS), lambda i: (0, 0)),
            pl.BlockSpec((1, UNITS), lambda i: (0, 0)),
            pl.BlockSpec((UNITS, 2 * UNITS), lambda i: (0, 0)),
            pl.BlockSpec((1, 2 * UNITS), lambda i: (0, 0)),
            pl.BlockSpec((2 * UNITS, 3 * UNITS), lambda i: (0, 0)),
            pl.BlockSpec((1, 3 * UNITS), lambda i: (0, 0)),
        ],
        out_specs=pl.BlockSpec((EDGE_BLK, 4 * UNITS), lambda i: (i, 0)),
        out_shape=jax.ShapeDtypeStruct((N_EDGES, 4 * UNITS), jnp.float32),
        compiler_params=pltpu.CompilerParams(
            allow_input_fusion=[True] + [False] * 7),
    )(edge_attr, edge_weight, w0, b0, w1, b1, w2p, b2p)


COLS_PER_CORE = 128
EPW = N_EDGES // 16
CHUNK = 128
NFULL = EPW // CHUNK
TAIL = EPW - NFULL * CHUNK
N_ACC = 10240
ROWS_PER_SUB = N_ACC // 16
LAST_ROWS = N_NODES - 15 * ROWS_PER_SUB


def _sc_segsum_body(h_hbm, dst_hbm, zeros_hbm, out_hbm,
                    idx0_v, idx1_v, rows0_v, rows1_v, idxt_v, rowst_v, acc_sh,
                    sem_i0, sem_i1, sem_r0, sem_r1):
    c = lax.axis_index("c")
    s = lax.axis_index("s")
    col0 = c * COLS_PER_CORE
    pltpu.sync_copy(zeros_hbm, acc_sh.at[pl.ds(s * ROWS_PER_SUB, ROWS_PER_SUB)])
    plsc.subcore_barrier()

    base = s * EPW
    idxb = (idx0_v, idx1_v)
    rowsb = (rows0_v, rows1_v)
    semi = (sem_i0, sem_i1)
    semr = (sem_r0, sem_r1)

    def start_load(chunk, b):
        off = base + chunk * CHUNK
        pltpu.async_copy(dst_hbm.at[pl.ds(off, CHUNK)], idxb[b], semi[b])
        pltpu.async_copy(
            h_hbm.at[pl.ds(off, CHUNK), pl.ds(col0, COLS_PER_CORE)],
            rowsb[b], semr[b])

    def wait_load(b):
        pltpu.make_async_copy(dst_hbm.at[pl.ds(0, CHUNK)], idxb[b],
                              semi[b]).wait()
        pltpu.make_async_copy(
            h_hbm.at[pl.ds(0, CHUNK), pl.ds(col0, COLS_PER_CORE)],
            rowsb[b], semr[b]).wait()

    start_load(0, 0)
    start_load(1, 1)

    def chunk_body(i, carry):
        for k in (0, 1):
            g = 2 * i + k
            wait_load(k)
            pltpu.sync_copy(rowsb[k], acc_sh.at[idxb[k]], add=True)

            @pl.when(g + 2 < NFULL)
            def _():
                start_load(g + 2, k)
        return carry

    lax.fori_loop(0, NFULL // 2, chunk_body, 0)

    b = base + NFULL * CHUNK
    pltpu.sync_copy(dst_hbm.at[pl.ds(b, TAIL)], idxt_v)
    pltpu.sync_copy(h_hbm.at[pl.ds(b, TAIL), pl.ds(col0, COLS_PER_CORE)],
                    rowst_v)
    pltpu.sync_copy(rowst_v, acc_sh.at[idxt_v], add=True)

    plsc.subcore_barrier()

    @pl.when(s < 15)
    def _():
        pltpu.sync_copy(acc_sh.at[pl.ds(s * ROWS_PER_SUB, ROWS_PER_SUB)],
                        out_hbm.at[pl.ds(s * ROWS_PER_SUB, ROWS_PER_SUB),
                                   pl.ds(col0, COLS_PER_CORE)])

    @pl.when(s == 15)
    def _():
        pltpu.sync_copy(acc_sh.at[pl.ds(15 * ROWS_PER_SUB, LAST_ROWS)],
                        out_hbm.at[pl.ds(15 * ROWS_PER_SUB, LAST_ROWS),
                                   pl.ds(col0, COLS_PER_CORE)])


def _sc_segsum(h, dst):
    zeros = jnp.zeros((ROWS_PER_SUB, COLS_PER_CORE), jnp.float32)
    mesh = plsc.VectorSubcoreMesh(core_axis_name="c", subcore_axis_name="s")
    f = pl.kernel(
        _sc_segsum_body,
        out_type=jax.ShapeDtypeStruct((N_NODES, 4 * UNITS), jnp.float32),
        mesh=mesh,
        scratch_types=[
            pltpu.VMEM((CHUNK,), jnp.int32),
            pltpu.VMEM((CHUNK,), jnp.int32),
            pltpu.VMEM((CHUNK, COLS_PER_CORE), jnp.float32),
            pltpu.VMEM((CHUNK, COLS_PER_CORE), jnp.float32),
            pltpu.VMEM((TAIL,), jnp.int32),
            pltpu.VMEM((TAIL, COLS_PER_CORE), jnp.float32),
            pltpu.VMEM_SHARED((N_ACC, COLS_PER_CORE), jnp.float32),
            pltpu.SemaphoreType.DMA,
            pltpu.SemaphoreType.DMA,
            pltpu.SemaphoreType.DMA,
            pltpu.SemaphoreType.DMA,
        ],
    )
    return f(h, dst, zeros)


NODE_BLK = 400

_IJ = [(0, 0), (0, 1), (0, 2), (1, 0), (1, 1), (1, 2), (2, 0), (2, 1), (2, 2)]
_UPPER = [(0, 1), (0, 2), (1, 2)]
_DIAG = [(0, 0), (1, 1), (2, 2)]
_SYM = [(0, 0), (0, 1), (0, 2), (1, 1), (1, 2), (2, 2)]


def _node_body(x_ref, h_ref, wt0_ref, wt1_ref, wt2_ref, wt3_ref, wt4_ref,
               wt5_ref, o_ref):
    xf = x_ref[...]
    x = [[xf[:, (3 * i + j) * UNITS:(3 * i + j + 1) * UNITS]
          for j in range(3)] for i in range(3)]
    norm = 1.0 + sum(x[i][j] * x[i][j] for (i, j) in _IJ)
    rn = 1.0 / norm
    xn = [[x[i][j] * rn for j in range(3)] for i in range(3)]

    i0 = (xn[0][0] + xn[1][1] + xn[2][2]) * (1.0 / 3.0)
    a = {(i, j): 0.5 * (xn[i][j] - xn[j][i]) for (i, j) in _UPPER}
    s = {(i, j): 0.5 * (xn[i][j] + xn[j][i]) for (i, j) in _UPPER}
    for (i, j) in _DIAG:
        s[(i, j)] = xn[i][j] - i0

    wt0 = wt0_ref[...]
    wt1 = wt1_ref[...]
    wt2 = wt2_ref[...]

    def mm(v, w):
        return jnp.dot(v, w, preferred_element_type=jnp.float32)

    il = mm(i0, wt0)
    al = {k: mm(a[k], wt1) for k in _UPPER}
    sl = {k: mm(s[k], wt2) for k in _SYM}

    def full_mat(diag_i, ad, sd):
        m = [[None] * 3 for _ in range(3)]
        for i in range(3):
            for j in range(3):
                if i == j:
                    m[i][j] = diag_i + sd[(i, i)]
                elif i < j:
                    m[i][j] = ad[(i, j)] + sd[(i, j)]
                else:
                    m[i][j] = sd[(j, i)] - ad[(j, i)]
        return m

    y = full_mat(il, al, sl)

    hsum = h_ref[...]
    h0 = hsum[:, 0:UNITS]
    h1 = hsum[:, UNITS:2 * UNITS]
    h2 = hsum[:, 2 * UNITS:3 * UNITS]

    ilh = il * h0
    alh = {k: al[k] * h1 for k in _UPPER}
    slh = {k: sl[k] * h2 for k in _SYM}
    msg = full_mat(ilh, alh, slh)

    c2 = [[None] * 3 for _ in range(3)]
    for i in range(3):
        for l in range(3):
            acc = y[i][0] * msg[0][l] + msg[i][0] * y[0][l]
            acc += y[i][1] * msg[1][l] + msg[i][1] * y[1][l]
            acc += y[i][2] * msg[2][l] + msg[i][2] * y[2][l]
            c2[i][l] = acc

    i2 = (c2[0][0] + c2[1][1] + c2[2][2]) * (1.0 / 3.0)
    np1 = 1.0 + sum(c2[i][j] * c2[i][j] for (i, j) in _IJ)
    rp = 1.0 / np1

    a2 = {(i, j): (0.5 * (c2[i][j] - c2[j][i])) * rp for (i, j) in _UPPER}
    s2 = {(i, j): (0.5 * (c2[i][j] + c2[j][i])) * rp for (i, j) in _UPPER}
    for (i, j) in _DIAG:
        s2[(i, j)] = (c2[i][j] - i2) * rp
    i2r = i2 * rp

    i3 = mm(i2r, wt3_ref[...])
    wt4 = wt4_ref[...]
    wt5 = wt5_ref[...]
    a3 = {k: mm(a2[k], wt4) for k in _UPPER}
    s3 = {k: mm(s2[k], wt5) for k in _SYM}
    dx = full_mat(i3, a3, s3)

    ocols = []
    for i in range(3):
        for j in range(3):
            o = xn[i][j] + dx[i][j]
            o += dx[i][0] * dx[0][j] + dx[i][1] * dx[1][j] + dx[i][2] * dx[2][j]
            ocols.append(o)
    o_ref[...] = jnp.concatenate(ocols, axis=1)


def _node_stage(x, hsum, wt0, wt1, wt2, wt3, wt4, wt5):
    grid = N_NODES // NODE_BLK
    wspec = pl.BlockSpec((UNITS, UNITS), lambda i: (0, 0))
    return pl.pallas_call(
        _node_body,
        grid=(grid,),
        in_specs=[
            pl.BlockSpec((NODE_BLK, 9 * UNITS), lambda i: (i, 0)),
            pl.BlockSpec((NODE_BLK, 4 * UNITS), lambda i: (i, 0)),
            wspec, wspec, wspec, wspec, wspec, wspec,
        ],
        out_specs=pl.BlockSpec((NODE_BLK, 9 * UNITS), lambda i: (i, 0)),
        out_shape=jax.ShapeDtypeStruct((N_NODES, 9 * UNITS), jnp.float32),
    )(x, hsum, wt0, wt1, wt2, wt3, wt4, wt5)


def kernel(X, edge_index, edge_weight, edge_attr, Ws0, bs0, Ws1, bs1, Ws2,
           bs2, Wt0, Wt1, Wt2, Wt3, Wt4, Wt5):
    perm = (jnp.arange(3)[:, None] + 3 * jnp.arange(UNITS)[None, :]).reshape(-1)
    w2p = Ws2[:, perm]
    b2p = bs2[perm]

    h = _edge_mlp(edge_attr, edge_weight.reshape(-1, EW_ROWS, 128),
                  Ws0.astype(jnp.bfloat16), bs0.reshape(1, -1),
                  Ws1.astype(jnp.bfloat16), bs1.reshape(1, -1),
                  w2p.astype(jnp.bfloat16), b2p.reshape(1, -1))
    dst = edge_index[1]
    hsum = _sc_segsum(h, dst)
    out = _node_stage(X.reshape(N_NODES, 9 * UNITS), hsum,
                      Wt0, Wt1, Wt2, Wt3, Wt4, Wt5)
    return out.reshape(N_NODES, 3, 3, UNITS)
